# baseline (speedup 1.0000x reference)
import jax
import jax.numpy as jnp
import numpy as np
from jax import lax
from jax.experimental import pallas as pl
from jax.experimental.pallas import tpu as pltpu

_P = 16
_NG = 24
_MASK_HEX = 0x3fb1ba87b53c66a55686f42a960ba650617a98fd1147f1ec95001871007dc656f12df699122ad5b25d199ac67833e13b0c42836153440def8d0f1e99c1e51d973eadcd2f8dc77f75


def _row_patterns_np() -> np.ndarray:
    bits = np.frombuffer(f"{_MASK_HEX:0576b}".encode(), np.uint8) - ord("0")
    m2 = bits.reshape(_NG, _NG)
    return (m2 << np.arange(_NG)[None, :]).sum(axis=1).astype(np.int32)


_PATTERNS_NP = _row_patterns_np()


def _select_body(pat_ref, x1_ref, x2_ref, o_ref):
    r = pl.program_id(1)
    pat = pat_ref[r]
    wp = lax.broadcasted_iota(jnp.int32, o_ref.shape, 2) // _P
    m = (pat >> wp) & 1
    o_ref[...] = jnp.where(m == 1, x2_ref[...], x1_ref[...])


def kernel(x1, x2):
    B, C, H, W = x1.shape
    N = B * C
    a = x1.reshape(N, H, W)
    b = x2.reshape(N, H, W)
    pats = jnp.asarray(_PATTERNS_NP)
    PB = 64
    out = pl.pallas_call(
        _select_body,
        grid=(N // PB, _NG),
        in_specs=[
            pl.BlockSpec(memory_space=pltpu.SMEM),
            pl.BlockSpec((PB, _P, W), lambda i, j: (i, j, 0)),
            pl.BlockSpec((PB, _P, W), lambda i, j: (i, j, 0)),
        ],
        out_specs=pl.BlockSpec((PB, _P, W), lambda i, j: (i, j, 0)),
        out_shape=jax.ShapeDtypeStruct((N, H, W), x1.dtype),
    )(pats, a, b)
    return out.reshape(B, C, H, W)


# speedup vs baseline: 1.0559x; 1.0559x over previous
import jax
import jax.numpy as jnp
import numpy as np
from jax.experimental import pallas as pl

_P = 16
_NG = 24


def _pixel_mask_np() -> np.ndarray:
    total = _NG * _NG
    rand_pos = np.asarray(jax.random.permutation(jax.random.key(42), total))
    m = np.zeros(total, np.bool_)
    m[rand_pos[: total // 2]] = True
    m2 = m.reshape(_NG, _NG)
    return np.repeat(np.repeat(m2, _P, axis=0), _P, axis=1)


_MASK_NP = _pixel_mask_np().astype(np.float32)


def _select_body(m_ref, x1_ref, x2_ref, o_ref):
    o_ref[...] = jnp.where(m_ref[...] != 0.0, x2_ref[...], x1_ref[...])


def kernel(x1, x2):
    B, C, H, W = x1.shape
    N = B * C
    a = x1.reshape(N, H, W)
    b = x2.reshape(N, H, W)
    m = jnp.asarray(_MASK_NP).reshape(1, H, W)
    PB = 4
    out = pl.pallas_call(
        _select_body,
        grid=(N // PB,),
        in_specs=[
            pl.BlockSpec((1, H, W), lambda i: (0, 0, 0)),
            pl.BlockSpec((PB, H, W), lambda i: (i, 0, 0)),
            pl.BlockSpec((PB, H, W), lambda i: (i, 0, 0)),
        ],
        out_specs=pl.BlockSpec((PB, H, W), lambda i: (i, 0, 0)),
        out_shape=jax.ShapeDtypeStruct((N, H, W), x1.dtype),
    )(m, a, b)
    return out.reshape(B, C, H, W)


# speedup vs baseline: 1.0773x; 1.0202x over previous
import jax
import jax.numpy as jnp
import numpy as np
from jax import lax
from jax.experimental import pallas as pl
from jax.experimental.pallas import tpu as pltpu

_P = 16
_NG = 24
_MASK_HEX = 0x3fb1ba87b53c66a55686f42a960ba650617a98fd1147f1ec95001871007dc656f12df699122ad5b25d199ac67833e13b0c42836153440def8d0f1e99c1e51d973eadcd2f8dc77f75


def _row_patterns_np() -> np.ndarray:
    bits = np.frombuffer(f"{_MASK_HEX:0576b}".encode(), np.uint8) - ord("0")
    m2 = bits.reshape(_NG, _NG)
    return (m2 << np.arange(_NG)[None, :]).sum(axis=1).astype(np.int32)


_PATTERNS_NP = _row_patterns_np()


def _select_body(pat_ref, x1_ref, x2_ref, o_ref, m_ref):
    @pl.when(pl.program_id(0) == 0)
    def _init_mask():
        wp = lax.broadcasted_iota(jnp.int32, (_P, _NG * _P), 1) // _P
        for r in range(_NG):
            m_ref[pl.ds(r * _P, _P), :] = (pat_ref[r] >> wp) & 1

    m = m_ref[...][None, :, :]
    o_ref[...] = jnp.where(m == 1, x2_ref[...], x1_ref[...])


def kernel(x1, x2):
    B, C, H, W = x1.shape
    N = B * C
    a = x1.reshape(N, H, W)
    b = x2.reshape(N, H, W)
    pats = jnp.asarray(_PATTERNS_NP)
    PB = 8
    out = pl.pallas_call(
        _select_body,
        grid=(N // PB,),
        in_specs=[
            pl.BlockSpec(memory_space=pltpu.SMEM),
            pl.BlockSpec((PB, H, W), lambda i: (i, 0, 0)),
            pl.BlockSpec((PB, H, W), lambda i: (i, 0, 0)),
        ],
        out_specs=pl.BlockSpec((PB, H, W), lambda i: (i, 0, 0)),
        out_shape=jax.ShapeDtypeStruct((N, H, W), x1.dtype),
        scratch_shapes=[pltpu.VMEM((H, W), jnp.int32)],
    )(pats, a, b)
    return out.reshape(B, C, H, W)
